# Initial kernel scaffold; baseline (speedup 1.0000x reference)
import jax
import jax.numpy as jnp
from jax.experimental import pallas as pl


def kernel(x, edge_index, W1, b1, W2, b2, W3, b3, W4, b4):
    raise NotImplementedError("write your pallas kernel here")


# speedup vs baseline: 2.7867x; 2.7867x over previous
import functools

import jax
import jax.numpy as jnp
from jax import lax
from jax.experimental import pallas as pl
from jax.experimental.pallas import tpu as pltpu
from jax.experimental.pallas import tpu_sc as plsc

N_NODES = 10000
D = 128
N_CORES = 2
N_SUBCORES = 16
NW = N_CORES * N_SUBCORES
N_PAD = 10240
RPT = N_PAD // N_SUBCORES
N_EDGES = 320000
CHUNK = 128
EPT = 10240
N_CHUNKS = EPT // CHUNK
E_PAD = NW * EPT

_mesh = plsc.VectorSubcoreMesh(core_axis_name="c", subcore_axis_name="s")


@functools.partial(
    pl.kernel,
    out_type=jax.ShapeDtypeStruct((N_CORES * N_PAD, D), jnp.float32),
    mesh=_mesh,
    scratch_types=[
        pltpu.VMEM((CHUNK,), jnp.int32),
        pltpu.VMEM((CHUNK,), jnp.int32),
        pltpu.VMEM((CHUNK, D), jnp.float32),
        pltpu.VMEM_SHARED((N_PAD, D), jnp.float32),
        pltpu.SemaphoreType.DMA,
    ],
)
def _sc_agg(f_hbm, src_hbm, dst_hbm, zeros_hbm, out_hbm,
            src_v, dst_v, rows_v, acc, sem):
    c = lax.axis_index("c")
    s = lax.axis_index("s")
    wid = c * N_SUBCORES + s

    pltpu.sync_copy(zeros_hbm, acc.at[pl.ds(s * RPT, RPT)])
    plsc.subcore_barrier()

    def body(i, carry):
        base = wid * EPT + i * CHUNK
        pltpu.sync_copy(src_hbm.at[pl.ds(base, CHUNK)], src_v)
        pltpu.sync_copy(dst_hbm.at[pl.ds(base, CHUNK)], dst_v)
        pltpu.async_copy(f_hbm.at[src_v], rows_v, sem).wait()
        pltpu.sync_copy(rows_v, acc.at[dst_v], add=True)
        return carry

    lax.fori_loop(0, N_CHUNKS, body, 0)
    plsc.subcore_barrier()

    pltpu.sync_copy(acc.at[pl.ds(s * RPT, RPT)],
                    out_hbm.at[pl.ds(c * N_PAD + s * RPT, RPT)])


def _dense_body(f_ref, agg_ref, w_ref, b_ref, o_ref, *, relu):
    h = f_ref[...] + agg_ref[0:N_PAD, :] + agg_ref[N_PAD:, :]
    o = jnp.dot(h, w_ref[...], preferred_element_type=jnp.float32) + b_ref[...]
    if relu:
        o = jnp.maximum(o, 0.0)
    o_ref[...] = o


def _dense(f, agg, w, b, relu):
    dout = w.shape[1]
    return pl.pallas_call(
        functools.partial(_dense_body, relu=relu),
        out_shape=jax.ShapeDtypeStruct((N_PAD, dout), jnp.float32),
    )(f, agg, w, b.reshape(1, dout))


def kernel(x, edge_index, W1, b1, W2, b2, W3, b3, W4, b4):
    src = edge_index[0].astype(jnp.int32)
    dst = edge_index[1].astype(jnp.int32)
    pad = E_PAD - N_EDGES
    src_p = jnp.concatenate([src, jnp.zeros((pad,), jnp.int32)])
    dst_p = jnp.concatenate([dst, jnp.full((pad,), N_NODES, jnp.int32)])
    f = jnp.concatenate(
        [x, jnp.zeros((N_PAD - N_NODES, D), jnp.float32)], axis=0)
    zeros_blk = jnp.zeros((RPT, D), jnp.float32)

    for w, b, relu in ((W1, b1, True), (W2, b2, True),
                       (W3, b3, True), (W4, b4, False)):
        agg = _sc_agg(f, src_p, dst_p, zeros_blk)
        f = _dense(f, agg, w, b, relu)
    return f[:N_NODES]


# speedup vs baseline: 4.8470x; 1.7393x over previous
import functools

import jax
import jax.numpy as jnp
from jax import lax
from jax.experimental import pallas as pl
from jax.experimental.pallas import tpu as pltpu
from jax.experimental.pallas import tpu_sc as plsc

N_NODES = 10000
D = 128
DH = 64
N_CORES = 2
N_SUBCORES = 16
NW = N_CORES * N_SUBCORES
N_PAD = 10240
RPT = N_PAD // N_SUBCORES
N_EDGES = 320000
CHUNK = 128
EPT = 20480
N_CHUNKS = EPT // CHUNK
E_PAD = N_SUBCORES * EPT
NBUF = 5
N_OUTER = N_CHUNKS // NBUF

_mesh = plsc.VectorSubcoreMesh(core_axis_name="c", subcore_axis_name="s")


@functools.partial(
    pl.kernel,
    out_type=jax.ShapeDtypeStruct((N_CORES * N_PAD, DH), jnp.float32),
    mesh=_mesh,
    scratch_types=[
        pltpu.VMEM((N_CHUNKS, CHUNK), jnp.int32),
        pltpu.VMEM((N_CHUNKS, CHUNK), jnp.int32),
        pltpu.VMEM((NBUF, CHUNK, DH), jnp.float32),
        pltpu.VMEM_SHARED((N_PAD, DH), jnp.float32),
        pltpu.SemaphoreType.DMA((NBUF,)),
        pltpu.SemaphoreType.DMA((NBUF,)),
    ],
    compiler_params=pltpu.CompilerParams(use_tc_tiling_on_sc=False),
)
def _sc_agg(f_hbm, src_hbm, dst_hbm, zeros_hbm, out_hbm,
            src_v, dst_v, rows, acc, gsem, ssem):
    c = lax.axis_index("c")
    s = lax.axis_index("s")
    wid = c * N_SUBCORES + s

    pltpu.sync_copy(zeros_hbm, rows.at[0])
    for r in range(RPT // CHUNK):
        pltpu.sync_copy(rows.at[0], acc.at[pl.ds(s * RPT + r * CHUNK, CHUNK)])

    pltpu.sync_copy(src_hbm.at[wid], src_v)
    pltpu.sync_copy(dst_hbm.at[s], dst_v)
    plsc.subcore_barrier()

    def gather_start(i, b):
        pltpu.async_copy(f_hbm.at[src_v.at[i]], rows.at[b], gsem.at[b])

    def gather_wait(i, b):
        pltpu.make_async_copy(f_hbm.at[src_v.at[i]], rows.at[b],
                              gsem.at[b]).wait()

    def scatter_start(i, b):
        pltpu.async_copy(rows.at[b], acc.at[dst_v.at[i]], ssem.at[b],
                         add=True)

    def scatter_wait(i, b):
        pltpu.make_async_copy(rows.at[b], acc.at[dst_v.at[i]],
                              ssem.at[b]).wait()

    for b in range(NBUF):
        gather_start(b, b)

    def outer(g, carry):
        i0 = g * NBUF
        for b in range(NBUF):
            gather_wait(i0 + b, b)
            scatter_start(i0 + b, b)
        for b in range(NBUF):
            i = i0 + b + NBUF

            @pl.when(i < N_CHUNKS)
            def _():
                scatter_wait(i0 + b, b)
                gather_start(i, b)

        return carry

    lax.fori_loop(0, N_OUTER, outer, 0)
    for b in range(NBUF):
        scatter_wait(N_CHUNKS - NBUF + b, b)
    plsc.subcore_barrier()

    pltpu.sync_copy(acc.at[pl.ds(s * RPT, RPT)],
                    out_hbm.at[pl.ds(c * N_PAD + s * RPT, RPT)])


def _dense_body(f_ref, agg_ref, w_ref, b_ref, o_ref, *, relu, split_out):
    h0 = f_ref[0:N_PAD, :] + agg_ref[0:N_PAD, :]
    h1 = f_ref[N_PAD:, :] + agg_ref[N_PAD:, :]
    h = jnp.concatenate([h0, h1], axis=1)
    o = jnp.dot(h, w_ref[...], preferred_element_type=jnp.float32) + b_ref[...]
    if relu:
        o = jnp.maximum(o, 0.0)
    if split_out:
        o_ref[0:N_PAD, :] = o[:, :DH]
        o_ref[N_PAD:, :] = o[:, DH:]
    else:
        o_ref[...] = o


def _dense(f, agg, w, b, relu, split_out):
    dout = w.shape[1]
    out_rows = N_CORES * N_PAD if split_out else N_PAD
    out_cols = DH if split_out else dout
    return pl.pallas_call(
        functools.partial(_dense_body, relu=relu, split_out=split_out),
        out_shape=jax.ShapeDtypeStruct((out_rows, out_cols), jnp.float32),
    )(f, agg, w, b.reshape(1, dout))


def kernel(x, edge_index, W1, b1, W2, b2, W3, b3, W4, b4):
    src = edge_index[0].astype(jnp.int32)
    dst = edge_index[1].astype(jnp.int32)
    pad = E_PAD - N_EDGES
    src_p = jnp.concatenate([src, jnp.zeros((pad,), jnp.int32)])
    dst_p = jnp.concatenate([dst, jnp.full((pad,), N_NODES, jnp.int32)])
    src2 = jnp.stack([src_p, src_p + N_PAD]).reshape(NW, N_CHUNKS, CHUNK)
    dst2 = dst_p.reshape(N_SUBCORES, N_CHUNKS, CHUNK)
    x_pad = jnp.concatenate(
        [x, jnp.zeros((N_PAD - N_NODES, D), jnp.float32)], axis=0)
    f = jnp.concatenate([x_pad[:, :DH], x_pad[:, DH:]], axis=0)
    zeros_blk = jnp.zeros((CHUNK, DH), jnp.float32)

    for w, b, relu, split in ((W1, b1, True, True), (W2, b2, True, True),
                              (W3, b3, True, True), (W4, b4, False, False)):
        agg = _sc_agg(f, src2, dst2, zeros_blk)
        f = _dense(f, agg, w, b, relu, split)
    return f[:N_NODES]
